# Initial kernel scaffold; baseline (speedup 1.0000x reference)
import jax
import jax.numpy as jnp
from jax.experimental import pallas as pl


def kernel(dense_input, sparse_input, W_dense, b_dense, W_sparse):
    raise NotImplementedError("write your pallas kernel here")


# speedup vs baseline: 1.2229x; 1.2229x over previous
import functools

import jax
import jax.numpy as jnp
from jax import lax
from jax.experimental import pallas as pl
from jax.experimental.pallas import tpu as pltpu
from jax.experimental.pallas import tpu_sc as plsc

BATCH = 16384
N_FIELDS = 26
LINEAR_SIZE = 13
LANES = 16
CHUNK = 128


def _sc_linear(table_hbm, idx_hbm, dense_hbm, wb_hbm, out_hbm,
               idx_v, vals_v, dense_v, wb_v, out_v, sem):
    info = plsc.get_sparse_core_info()
    nc, ns = info.num_cores, info.num_subcores
    nw = nc * ns
    rows = BATCH // nw
    n_idx = rows * N_FIELDS
    n_chunks = n_idx // CHUNK

    wid = lax.axis_index("s") * nc + lax.axis_index("c")
    base = wid * rows

    pltpu.sync_copy(idx_hbm.at[wid], idx_v)
    pltpu.sync_copy(dense_hbm.at[wid], dense_v)
    pltpu.sync_copy(wb_hbm, wb_v)

    def fire(c, carry):
        off = pl.multiple_of(c * CHUNK, CHUNK)
        pltpu.make_async_copy(
            table_hbm.at[idx_v.at[pl.ds(off, CHUNK)]],
            vals_v.at[pl.ds(off, CHUNK)],
            sem,
        ).start()
        return carry

    lax.fori_loop(0, n_chunks, fire, 0)

    def drain(c, carry):
        off = pl.multiple_of(c * CHUNK, CHUNK)
        pltpu.make_async_copy(
            table_hbm.at[idx_v.at[pl.ds(off, CHUNK)]],
            vals_v.at[pl.ds(off, CHUNK)],
            sem,
        ).wait()
        return carry

    lax.fori_loop(0, n_chunks, drain, 0)

    wvec = wb_v[...]
    w = [wvec[j] for j in range(LINEAR_SIZE)]
    b = wvec[LINEAR_SIZE]

    def body(g, carry):
        goff = pl.multiple_of(g * LANES, LANES)
        acc = jnp.full((LANES,), b, dtype=jnp.float32)
        for j in range(LINEAR_SIZE):
            acc = acc + dense_v[pl.ds(j * rows + goff, LANES)] * w[j]
        for f in range(N_FIELDS):
            acc = acc + vals_v[pl.ds(f * rows + goff, LANES)]
        out_v[pl.ds(goff, LANES)] = acc
        return carry

    lax.fori_loop(0, rows // LANES, body, 0)

    pltpu.sync_copy(out_v, out_hbm.at[pl.ds(base, rows)])


def kernel(dense_input, sparse_input, W_dense, b_dense, W_sparse):
    info = plsc.get_sparse_core_info()
    nw = info.num_cores * info.num_subcores
    rows = BATCH // nw
    n_idx = rows * N_FIELDS

    idx2 = (sparse_input.astype(jnp.int32)
            .reshape(nw, rows, N_FIELDS).transpose(0, 2, 1).reshape(nw, n_idx))
    dense2 = (dense_input.reshape(nw, rows, LINEAR_SIZE)
              .transpose(0, 2, 1).reshape(nw, rows * LINEAR_SIZE))
    wb = jnp.concatenate(
        [W_dense.reshape(-1), b_dense.reshape(-1),
         jnp.zeros((LANES - LINEAR_SIZE - 1,), jnp.float32)])
    table = W_sparse.reshape(-1)

    mesh = plsc.VectorSubcoreMesh(core_axis_name="c", subcore_axis_name="s")
    run = pl.kernel(
        _sc_linear,
        mesh=mesh,
        out_type=jax.ShapeDtypeStruct((BATCH,), jnp.float32),
        scratch_types=[
            pltpu.VMEM((n_idx,), jnp.int32),
            pltpu.VMEM((n_idx,), jnp.float32),
            pltpu.VMEM((rows * LINEAR_SIZE,), jnp.float32),
            pltpu.VMEM((LANES,), jnp.float32),
            pltpu.VMEM((rows,), jnp.float32),
            pltpu.SemaphoreType.DMA,
        ],
    )
    out = run(table, idx2, dense2, wb)
    return out.reshape(BATCH, 1)
